# baseline (speedup 1.0000x reference)
import functools

import jax
import jax.numpy as jnp
from jax import lax
from jax.experimental import pallas as pl
from jax.experimental.pallas import tpu as pltpu
from jax.experimental.pallas import tpu_sc as plsc

PASS_TIME = 1.0
N_INTERVAL = 100
OUT_DIM = 64
NBINS = N_INTERVAL + 1

NW = 32
CHUNK = 512
NGROUP = CHUNK // 16


def _sc_time_encode(ts_a, ts_b, table_flat, rows):
    rpw = rows // NW
    nchunks = rpw // CHUNK
    npairs = nchunks // 2
    tabn = NBINS * OUT_DIM
    mesh = plsc.VectorSubcoreMesh(core_axis_name="c", subcore_axis_name="s")

    @functools.partial(
        pl.kernel,
        mesh=mesh,
        out_type=jax.ShapeDtypeStruct((rows * OUT_DIM,), jnp.float32),
        scratch_types=[
            pltpu.VMEM((rpw,), jnp.float32),
            pltpu.VMEM((rpw,), jnp.float32),
            pltpu.VMEM((tabn,), jnp.float32),
            pltpu.VMEM((CHUNK * OUT_DIM,), jnp.float32),
            pltpu.VMEM((CHUNK * OUT_DIM,), jnp.float32),
            pltpu.SemaphoreType.DMA,
            pltpu.SemaphoreType.DMA,
        ],
        compiler_params=pltpu.CompilerParams(needs_layout_passes=False),
    )
    def k(a_h, b_h, tab_h, out_h, a_v, b_v, tab_v, bufa, bufb, sema, semb):
        wid = lax.axis_index("s") * 2 + lax.axis_index("c")
        base = pl.multiple_of(wid * rpw, rpw)
        pltpu.sync_copy(a_h.at[pl.ds(base, rpw)], a_v)
        pltpu.sync_copy(b_h.at[pl.ds(base, rpw)], b_v)
        pltpu.sync_copy(tab_h, tab_v)
        iota = lax.iota(jnp.int32, 16)
        row64 = iota * OUT_DIM
        offlow = [(iota + i) & 15 for i in range(16)]

        def assemble(c_idx, buf):
            roff = pl.multiple_of(c_idx * CHUNK, CHUNK)

            def group(g, carry):
                s = pl.ds(pl.multiple_of(roff + g * 16, 16), 16)
                dt = b_v[s] - a_v[s]
                q = (dt * (N_INTERVAL / PASS_TIME)).astype(jnp.int32)
                idx = jnp.minimum(jnp.maximum(q, 0), N_INTERVAL)
                src = idx * OUT_DIM
                dst = row64 + g * (16 * OUT_DIM)
                nbuf = CHUNK * OUT_DIM
                for c0 in range(0, OUT_DIM, 16):
                    tslice = tab_v.at[pl.ds(c0, tabn - 48)]
                    bslice = buf.at[pl.ds(c0, nbuf - 48)]
                    for i0 in range(0, 16, 8):
                        vals = [
                            plsc.load_gather(tslice, [src + offlow[i0 + i]])
                            for i in range(8)
                        ]
                        for i in range(8):
                            plsc.store_scatter(
                                bslice, [dst + offlow[i0 + i]], vals[i]
                            )
                return carry

            lax.fori_loop(0, NGROUP, group, 0)

        def start_store(c_idx, buf, sem):
            off = pl.multiple_of((base + c_idx * CHUNK) * OUT_DIM, CHUNK * OUT_DIM)
            return pltpu.async_copy(
                buf, out_h.at[pl.ds(off, CHUNK * OUT_DIM)], sem
            )

        def wait_store(buf, sem):
            pltpu.make_async_copy(
                buf, out_h.at[pl.ds(base * OUT_DIM, CHUNK * OUT_DIM)], sem
            ).wait()

        assemble(0, bufa)
        start_store(0, bufa, sema)
        assemble(1, bufb)
        start_store(1, bufb, semb)

        def pair(p, carry):
            wait_store(bufa, sema)
            assemble(2 * p, bufa)
            start_store(2 * p, bufa, sema)
            wait_store(bufb, semb)
            assemble(2 * p + 1, bufb)
            start_store(2 * p + 1, bufb, semb)
            return carry

        lax.fori_loop(1, npairs, pair, 0)
        wait_store(bufa, sema)
        wait_store(bufb, semb)

    return k(ts_a, ts_b, table_flat)


def _tc_slice(timestamp):
    batch, l1 = timestamp.shape
    L = l1 - 1
    blk = 512

    def body(ts_ref, a_ref, b_ref):
        a_ref[...] = ts_ref[:, :L]
        b_ref[...] = ts_ref[:, 1:]

    return pl.pallas_call(
        body,
        grid=(batch // blk,),
        in_specs=[pl.BlockSpec((blk, l1), lambda i: (i, 0))],
        out_specs=[pl.BlockSpec((blk, L), lambda i: (i, 0))] * 2,
        out_shape=[jax.ShapeDtypeStruct((batch, L), jnp.float32)] * 2,
    )(timestamp)


def kernel(inputs, timestamp, train, W, b):
    batch, L = inputs.shape
    rows = batch * L
    table = (W.T + b[None, :]).reshape(NBINS * OUT_DIM)
    ts_prev, ts_next = _tc_slice(timestamp)
    ts_a = ts_prev.reshape(rows)
    ts_b = ts_next.reshape(rows)
    out = _sc_time_encode(ts_a, ts_b, table, rows)
    return out.reshape(batch, L, OUT_DIM), ts_prev
